# Initial kernel scaffold; baseline (speedup 1.0000x reference)
import jax
import jax.numpy as jnp
from jax.experimental import pallas as pl


def kernel(indices, emb, W):
    raise NotImplementedError("write your pallas kernel here")


# speedup vs baseline: 192.6207x; 192.6207x over previous
import functools

import jax
import jax.numpy as jnp
from jax import lax
from jax.experimental import pallas as pl
from jax.experimental.pallas import tpu as pltpu
from jax.experimental.pallas import tpu_sc as plsc

B = 16384
L = 200
NC = 2
NS = 16
NW = NC * NS
RPW = B // NW
CHUNK = 16
NCHUNK = RPW // CHUNK


def _body(idx_hbm, emb_hbm, wt_hbm, out_hbm, ev, wv, t8, idx_v, out_v, sem):
    del sem
    wid = lax.axis_index("s") * NC + lax.axis_index("c")
    base = wid * RPW

    pltpu.sync_copy(emb_hbm, ev)
    pltpu.sync_copy(wt_hbm, wv)
    w = wv[...]
    lane = lax.iota(jnp.int32, 16)
    tvec = jnp.zeros((16,), jnp.float32)
    for half in range(2):
        p = ev[pl.ds(16 * half, 16)] * w
        for r in range(4):
            m = (lane >= 4 * r) & (lane < 4 * r + 4)
            s = jnp.sum(jnp.where(m, p, 0.0))
            tvec = jnp.where(lane == (half * 4 + r), s, tvec)
    t8[...] = tvec

    def chunk_body(c, carry):
        row0 = base + c * CHUNK
        pltpu.sync_copy(idx_hbm.at[pl.ds(row0, CHUNK)], idx_v)
        ovec = jnp.zeros((16,), jnp.float32)
        for r in range(CHUNK):
            acc = jnp.zeros((16,), jnp.float32)
            for j in range(12):
                ii = idx_v[r, pl.ds(16 * j, 16)]
                acc = acc + plsc.load_gather(t8, [ii])
            ii = idx_v[r, pl.ds(L - 16, 16)]
            g = plsc.load_gather(t8, [ii])
            acc = acc + jnp.where(lane >= 8, g, 0.0)
            ovec = jnp.where(lane == r, jnp.sum(acc), ovec)
        out_v[...] = ovec
        pltpu.sync_copy(out_v, out_hbm.at[pl.ds(row0, CHUNK)])
        return carry

    lax.fori_loop(0, NCHUNK, chunk_body, 0)


@jax.jit
def _run(indices, emb_flat, wt):
    mesh = plsc.VectorSubcoreMesh(core_axis_name="c", subcore_axis_name="s")
    f = pl.kernel(
        _body,
        out_type=jax.ShapeDtypeStruct((B,), jnp.float32),
        mesh=mesh,
        compiler_params=pltpu.CompilerParams(needs_layout_passes=False),
        scratch_types=[
            pltpu.VMEM((32,), jnp.float32),
            pltpu.VMEM((16,), jnp.float32),
            pltpu.VMEM((16,), jnp.float32),
            pltpu.VMEM((CHUNK, L), jnp.int32),
            pltpu.VMEM((16,), jnp.float32),
            pltpu.SemaphoreType.DMA,
        ],
    )
    return f(indices, emb_flat, wt)


def kernel(indices, emb, W):
    emb_flat = emb.reshape(32)
    wt = jnp.tile(W.reshape(4), 4)
    out = _run(indices, emb_flat, wt)
    return out.reshape(B, 1)


# speedup vs baseline: 270.3230x; 1.4034x over previous
import functools

import jax
import jax.numpy as jnp
from jax import lax
from jax.experimental import pallas as pl
from jax.experimental.pallas import tpu as pltpu
from jax.experimental.pallas import tpu_sc as plsc

B = 16384
L = 200
NC = 2
NS = 16
NW = NC * NS
RPW = B // NW
CHUNK = 64
NCHUNK = RPW // CHUNK
GROUPS = CHUNK // 16


def _body(idx_hbm, emb_hbm, wt_hbm, out_hbm,
          ev, wv, t8, t64, t512, ibuf0, ibuf1, obuf0, obuf1,
          isem0, isem1, osem0, osem1):
    wid = lax.axis_index("s") * NC + lax.axis_index("c")
    base = wid * RPW
    lane = lax.iota(jnp.int32, 16)

    pltpu.async_copy(idx_hbm.at[pl.ds(base, CHUNK)], ibuf0, isem0)
    pltpu.async_copy(idx_hbm.at[pl.ds(base + CHUNK, CHUNK)], ibuf1, isem1)

    pltpu.sync_copy(emb_hbm, ev)
    pltpu.sync_copy(wt_hbm, wv)
    w = wv[...]
    tvec = jnp.zeros((16,), jnp.float32)
    for half in range(2):
        p = ev[pl.ds(16 * half, 16)] * w
        for r in range(4):
            m = (lane >= 4 * r) & (lane < 4 * r + 4)
            s = jnp.sum(jnp.where(m, p, 0.0))
            tvec = jnp.where(lane == (half * 4 + r), s, tvec)
    t8[...] = tvec

    for m in range(4):
        q = lane + 16 * m
        t64[pl.ds(16 * m, 16)] = (plsc.load_gather(t8, [q >> 3]) +
                                  plsc.load_gather(t8, [q & 7]))
    for m in range(32):
        q = lane + 16 * m
        t512[pl.ds(16 * m, 16)] = (plsc.load_gather(t64, [q >> 3]) +
                                   plsc.load_gather(t8, [q & 7]))

    bufs = ((ibuf0, obuf0, isem0, osem0), (ibuf1, obuf1, isem1, osem1))

    @pl.loop(0, NCHUNK, step=2)
    def chunk_loop(c0):
        for bsel in range(2):
            ibuf, obuf, isem, osem = bufs[bsel]
            c = c0 + bsel
            row0 = base + c * CHUNK
            pltpu.make_async_copy(idx_hbm.at[pl.ds(base, CHUNK)], ibuf,
                                  isem).wait()

            @pl.when(c0 >= 2)
            def _wait_out():
                pltpu.make_async_copy(obuf, out_hbm.at[pl.ds(base, CHUNK)],
                                      osem).wait()

            def group(g, carry):
                ovec = jnp.zeros((16,), jnp.float32)
                for ri in range(16):
                    r = g * 16 + ri
                    acc = jnp.zeros((16,), jnp.float32)
                    for gq in range(4):
                        i0 = ibuf[r, pl.ds(48 * gq, 16)]
                        i1 = ibuf[r, pl.ds(48 * gq + 16, 16)]
                        i2 = ibuf[r, pl.ds(48 * gq + 32, 16)]
                        comb = i0 + (i1 << 3) + (i2 << 6)
                        acc = acc + plsc.load_gather(t512, [comb])
                    ii = ibuf[r, pl.ds(L - 16, 16)]
                    g8 = plsc.load_gather(t8, [ii])
                    acc = acc + jnp.where(lane >= 8, g8, 0.0)
                    ovec = jnp.where(lane == ri, jnp.sum(acc), ovec)
                obuf[pl.ds(g * 16, 16)] = ovec
                return carry

            lax.fori_loop(0, GROUPS, group, 0)
            pltpu.async_copy(obuf, out_hbm.at[pl.ds(row0, CHUNK)], osem)

            @pl.when(c + 2 < NCHUNK)
            def _prefetch():
                pltpu.async_copy(
                    idx_hbm.at[pl.ds(base + (c + 2) * CHUNK, CHUNK)],
                    ibuf, isem)

    pltpu.make_async_copy(obuf0, out_hbm.at[pl.ds(base, CHUNK)], osem0).wait()
    pltpu.make_async_copy(obuf1, out_hbm.at[pl.ds(base, CHUNK)], osem1).wait()


@jax.jit
def _run(indices, emb_flat, wt):
    mesh = plsc.VectorSubcoreMesh(core_axis_name="c", subcore_axis_name="s")
    f = pl.kernel(
        _body,
        out_type=jax.ShapeDtypeStruct((B,), jnp.float32),
        mesh=mesh,
        compiler_params=pltpu.CompilerParams(needs_layout_passes=False),
        scratch_types=[
            pltpu.VMEM((32,), jnp.float32),
            pltpu.VMEM((16,), jnp.float32),
            pltpu.VMEM((16,), jnp.float32),
            pltpu.VMEM((64,), jnp.float32),
            pltpu.VMEM((512,), jnp.float32),
            pltpu.VMEM((CHUNK, L), jnp.int32),
            pltpu.VMEM((CHUNK, L), jnp.int32),
            pltpu.VMEM((CHUNK,), jnp.float32),
            pltpu.VMEM((CHUNK,), jnp.float32),
            pltpu.SemaphoreType.DMA,
            pltpu.SemaphoreType.DMA,
            pltpu.SemaphoreType.DMA,
            pltpu.SemaphoreType.DMA,
        ],
    )
    return f(indices, emb_flat, wt)


def kernel(indices, emb, W):
    emb_flat = emb.reshape(32)
    wt = jnp.tile(W.reshape(4), 4)
    out = _run(indices, emb_flat, wt)
    return out.reshape(B, 1)
